# baseline (speedup 1.0000x reference)
import jax
import jax.numpy as jnp
from jax.experimental import pallas as pl
from jax.experimental.pallas import tpu as pltpu

_COLS = 1024
_ROWS_PER_BLOCK = 256


def _body(yp_ref, s_ref, g_ref, out_ref, acc_ref):
    i = pl.program_id(0)
    k = pl.num_programs(0)

    @pl.when(i == 0)
    def _init():
        acc_ref[0] = jnp.float32(0.0)
        acc_ref[1] = jnp.float32(0.0)
        acc_ref[2] = jnp.float32(0.0)
        acc_ref[3] = jnp.float32(0.0)

    yp = yp_ref[...]
    sv = s_ref[...]
    gv = g_ref[...]
    pos = gv == 1
    m0 = pos & (sv == 0)
    m1 = pos & (sv == 1)
    zero = jnp.float32(0.0)
    one = jnp.float32(1.0)
    acc_ref[0] += jnp.sum(jnp.where(m0, yp, zero))
    acc_ref[1] += jnp.sum(jnp.where(m1, yp, zero))
    acc_ref[2] += jnp.sum(jnp.where(m0, one, zero))
    acc_ref[3] += jnp.sum(jnp.where(m1, one, zero))

    @pl.when(i == k - 1)
    def _fini():
        sum0 = acc_ref[0]
        sum1 = acc_ref[1]
        n0 = acc_ref[2]
        n1 = acc_ref[3]
        mean0 = sum0 / jnp.maximum(n0, jnp.float32(1.0))
        mean1 = sum1 / jnp.maximum(n1, jnp.float32(1.0))
        loss = jnp.abs(mean0 - mean1)
        out_ref[0] = jnp.where((n0 == 0.0) | (n1 == 0.0), jnp.float32(0.0), loss)


def kernel(y_pred, s, y_gt):
    n = y_pred.size
    rows = n // _COLS
    grid = rows // _ROWS_PER_BLOCK
    yp = y_pred.reshape(rows, _COLS)
    sv = s.astype(jnp.int32).reshape(rows, _COLS)
    gv = y_gt.astype(jnp.int32).reshape(rows, _COLS)

    in_spec = pl.BlockSpec((_ROWS_PER_BLOCK, _COLS), lambda i: (i, 0))
    out = pl.pallas_call(
        _body,
        grid=(grid,),
        in_specs=[in_spec, in_spec, in_spec],
        out_specs=pl.BlockSpec(memory_space=pltpu.SMEM),
        out_shape=jax.ShapeDtypeStruct((1,), jnp.float32),
        scratch_shapes=[pltpu.SMEM((4,), jnp.float32)],
    )(yp, sv, gv)
    return out[0]
